# baseline (speedup 1.0000x reference)
import functools

import jax
import jax.numpy as jnp
from jax import lax
from jax.experimental import pallas as pl
from jax.experimental.pallas import tpu as pltpu
from jax.experimental.pallas import tpu_sc as plsc

LANES = 16


@functools.lru_cache(maxsize=None)
def _build(n_rows: int, vocab: int, embed: int, seq_len: int, chunk: int):
    info = plsc.get_sparse_core_info()
    nw = info.num_cores * info.num_subcores
    assert n_rows % nw == 0
    rows_per_w = n_rows // nw
    assert rows_per_w % chunk == 0
    n_chunks = rows_per_w // chunk

    mesh = plsc.VectorSubcoreMesh(core_axis_name="c", subcore_axis_name="s")

    @functools.partial(
        pl.kernel,
        out_type=jax.ShapeDtypeStruct((n_rows, embed), jnp.float32),
        mesh=mesh,
        scratch_types=[
            pltpu.VMEM((chunk,), jnp.int32),
            pltpu.VMEM((chunk, embed), jnp.float32),
            pltpu.SemaphoreType.DMA,
            pltpu.SemaphoreType.DMA,
        ],
        compiler_params=pltpu.CompilerParams(use_tc_tiling_on_sc=False),
    )
    def emb_kernel(table_hbm, idx_hbm, posrep_hbm, out_hbm,
                   idx_v, rows_v, gsem, psem):
        wid = lax.axis_index("s") * info.num_cores + lax.axis_index("c")
        base = wid * rows_per_w

        def chunk_body(c, _):
            off = base + c * chunk
            pltpu.sync_copy(idx_hbm.at[pl.ds(off, chunk)], idx_v)
            pltpu.async_copy(posrep_hbm, rows_v, psem).wait()
            pltpu.async_copy(table_hbm.at[idx_v], rows_v, gsem, add=True).wait()
            pltpu.sync_copy(rows_v, out_hbm.at[pl.ds(off, chunk)])
            return _

        lax.fori_loop(0, n_chunks, chunk_body, None)

    return emb_kernel


def kernel(inputs, token_table, pos_table):
    batch, seq_len = inputs.shape
    vocab, embed = token_table.shape
    flat_idx = inputs.reshape(-1).astype(jnp.int32)
    chunk = 2 * seq_len
    pos_rep = jnp.tile(pos_table, (chunk // seq_len, 1))
    fn = _build(batch * seq_len, vocab, embed, seq_len, chunk)
    out = fn(token_table, flat_idx, pos_rep)
    return out.reshape(batch, seq_len, embed)


# speedup vs baseline: 1.4936x; 1.4936x over previous
import functools

import jax
import jax.numpy as jnp
from jax import lax
from jax.experimental import pallas as pl
from jax.experimental.pallas import tpu as pltpu
from jax.experimental.pallas import tpu_sc as plsc

LANES = 16


@functools.lru_cache(maxsize=None)
def _build(n_rows: int, vocab: int, embed: int, seq_len: int):
    info = plsc.get_sparse_core_info()
    nw = info.num_cores * info.num_subcores
    chunk = 2 * seq_len
    assert n_rows % nw == 0
    rows_per_w = n_rows // nw
    assert rows_per_w % chunk == 0
    n_chunks = rows_per_w // chunk
    assert n_chunks % 2 == 0
    vecs = embed // LANES

    mesh = plsc.VectorSubcoreMesh(core_axis_name="c", subcore_axis_name="s")

    @functools.partial(
        pl.kernel,
        out_type=jax.ShapeDtypeStruct((n_rows, embed), jnp.float32),
        mesh=mesh,
        scratch_types=[
            pltpu.VMEM((chunk,), jnp.int32),
            pltpu.VMEM((chunk,), jnp.int32),
            pltpu.VMEM((chunk, embed), jnp.float32),
            pltpu.VMEM((chunk, embed), jnp.float32),
            pltpu.VMEM((chunk, embed), jnp.float32),
            pltpu.VMEM((chunk, embed), jnp.float32),
            pltpu.VMEM((seq_len, embed), jnp.float32),
            pltpu.SemaphoreType.DMA,
            pltpu.SemaphoreType.DMA,
            pltpu.SemaphoreType.DMA,
            pltpu.SemaphoreType.DMA,
        ],
        compiler_params=pltpu.CompilerParams(use_tc_tiling_on_sc=False),
    )
    def emb_kernel(table_hbm, idx_hbm, pos_hbm, out_hbm,
                   idx0, idx1, rows0, rows1, ob0, ob1, pos_v,
                   gsem0, gsem1, wsem0, wsem1):
        idx_v = (idx0, idx1)
        rows_v = (rows0, rows1)
        outb = (ob0, ob1)
        gsem = (gsem0, gsem1)
        wsem = (wsem0, wsem1)

        wid = lax.axis_index("s") * info.num_cores + lax.axis_index("c")
        base = wid * rows_per_w

        pltpu.sync_copy(pos_hbm, pos_v)

        for b in range(2):
            pltpu.sync_copy(idx_hbm.at[pl.ds(base + b * chunk, chunk)], idx_v[b])
            pltpu.async_copy(table_hbm.at[idx_v[b]], rows_v[b], gsem[b])

        def pair_body(i, _):
            for b in range(2):
                c = 2 * i + b
                pltpu.make_async_copy(
                    table_hbm.at[idx_v[b]], rows_v[b], gsem[b]).wait()
                off2 = base + lax.min(c + 2, n_chunks - 1) * chunk
                pltpu.sync_copy(idx_hbm.at[pl.ds(off2, chunk)], idx_v[b])
                @pl.when(i > 0)
                def _wait_prev_write():
                    pltpu.make_async_copy(
                        outb[b], out_hbm.at[pl.ds(base, chunk)], wsem[b]).wait()

                def add_body(s, _):
                    for k in range(vecs):
                        ds = pl.ds(k * LANES, LANES)
                        p = pos_v[s, ds]
                        outb[b][s, ds] = rows_v[b][s, ds] + p
                        outb[b][seq_len + s, ds] = rows_v[b][seq_len + s, ds] + p
                    return _

                lax.fori_loop(0, seq_len, add_body, None)
                pltpu.async_copy(table_hbm.at[idx_v[b]], rows_v[b], gsem[b])
                pltpu.async_copy(
                    outb[b], out_hbm.at[pl.ds(base + c * chunk, chunk)], wsem[b])
            return _

        lax.fori_loop(0, n_chunks // 2, pair_body, None)

        for b in range(2):
            pltpu.make_async_copy(
                table_hbm.at[idx_v[b]], rows_v[b], gsem[b]).wait()
            pltpu.make_async_copy(
                outb[b], out_hbm.at[pl.ds(base, chunk)], wsem[b]).wait()

    return emb_kernel


def kernel(inputs, token_table, pos_table):
    batch, seq_len = inputs.shape
    vocab, embed = token_table.shape
    flat_idx = inputs.reshape(-1).astype(jnp.int32)
    fn = _build(batch * seq_len, vocab, embed, seq_len)
    out = fn(token_table, flat_idx, pos_table)
    return out.reshape(batch, seq_len, embed)
